# baseline (speedup 1.0000x reference)
import functools

import jax
import jax.numpy as jnp
from jax import lax
from jax.experimental import pallas as pl
from jax.experimental.pallas import tpu as pltpu
from jax.experimental.pallas import tpu_sc as plsc

F = 26
V = 100000
D = 32
B = 16384

NC = 2
NS = 16
NW = NC * NS
L = 16
HV = 50048
W1 = 49920
WT = 99968
ICH = 4096
NICH = B // ICH


@functools.partial(
    pl.kernel,
    mesh=plsc.VectorSubcoreMesh(core_axis_name="c", subcore_axis_name="s"),
    out_type=jax.ShapeDtypeStruct((D, B), jnp.float32),
    scratch_types=[
        pltpu.VMEM((HV,), jnp.float32),
        pltpu.VMEM((50048 + 32,), jnp.float32),
        pltpu.VMEM((B,), jnp.float32),
        pltpu.VMEM((ICH,), jnp.int32),
        pltpu.VMEM((ICH,), jnp.int32),
        pltpu.SemaphoreType.DMA,
        pltpu.SemaphoreType.DMA,
        pltpu.SemaphoreType.DMA,
        pltpu.SemaphoreType.DMA,
    ],
    compiler_params=pltpu.CompilerParams(needs_layout_passes=False),
)
def _emb_lookup_sum(tabfd, idxT, outT, r0, r1, acc, ib0, ib1,
                    semr0, semr1, semi0, semi1):
    d = lax.axis_index("s") * NC + lax.axis_index("c")

    def zero(i, _):
        acc[pl.ds(i * L, L)] = jnp.zeros((L,), jnp.float32)
        return 0

    lax.fori_loop(0, B // L, zero, 0, unroll=8)

    rows = (r0, r1)
    semrs = (semr0, semr1)
    ibs = (ib0, ib1)
    semis = (semi0, semi1)

    def issue_row(f, h):
        rowref = tabfd.at[f * D + d]
        if h == 0:
            pltpu.async_copy(rowref.at[pl.ds(0, HV)], r0, semr0)
        else:
            pltpu.async_copy(rowref.at[pl.ds(W1, V - W1)], r1, semr1)

    def drain_row(h):
        if h == 0:
            pltpu.make_async_copy(
                tabfd.at[0].at[pl.ds(0, HV)], r0, semr0).wait()
        else:
            pltpu.make_async_copy(
                tabfd.at[0].at[pl.ds(W1, V - W1)], r1, semr1).wait()

    issue_row(0, 0)
    issue_row(0, 1)

    def step_pair(f, _):
        for h in range(2):
            rbuf = rows[h]
            lbase = h * W1
            mbase = h * HV
            mlen = HV if h == 0 else V - HV

            drain_row(h)

            pltpu.async_copy(idxT.at[f, pl.ds(0, ICH)], ib0, semi0)

            for c in range(NICH):
                p = c % 2
                q = 1 - p
                pltpu.make_async_copy(
                    idxT.at[f, pl.ds(0, ICH)], ibs[p], semis[p]).wait()
                if c + 1 < NICH:
                    pltpu.async_copy(
                        idxT.at[f, pl.ds((c + 1) * ICH, ICH)], ibs[q],
                        semis[q])
                ib = ibs[p]
                cbase = c * ICH

                def body(r, _, ib=ib, cbase=cbase, lbase=lbase,
                         mbase=mbase, mlen=mlen, rbuf=rbuf):
                    iv = ib[pl.ds(r * L, L)]
                    mask = (iv - mbase).astype(jnp.uint32) < jnp.uint32(mlen)
                    loc = iv - lbase
                    g = plsc.load_gather(rbuf, [loc], mask=mask)
                    g = jnp.where(mask, g, jnp.float32(0.0))
                    plsc.addupdate(acc.at[pl.ds(cbase + r * L, L)], g)
                    return 0

                lax.fori_loop(0, ICH // L, body, 0, unroll=8)

            @pl.when(f + 1 < F)
            def _(f=f, h=h):
                issue_row(f + 1, h)

        return 0

    lax.fori_loop(0, F, step_pair, 0)
    pltpu.sync_copy(acc, outT.at[d])


def kernel(x, tables):
    x = x.astype(jnp.int32)
    xT = x.T
    tabfd = tables.transpose(0, 2, 1).reshape(F * D, V)
    outT = _emb_lookup_sum(tabfd, xT)
    return outT.T


# speedup vs baseline: 1.0007x; 1.0007x over previous
import functools

import jax
import jax.numpy as jnp
from jax import lax
from jax.experimental import pallas as pl
from jax.experimental.pallas import tpu as pltpu
from jax.experimental.pallas import tpu_sc as plsc

F = 26
V = 100000
D = 32
B = 16384

NC = 2
NS = 16
NW = NC * NS
L = 16
HV = 50048
ICH = 4096
NICH = B // ICH


@functools.partial(
    pl.kernel,
    mesh=plsc.VectorSubcoreMesh(core_axis_name="c", subcore_axis_name="s"),
    out_type=jax.ShapeDtypeStruct((D, B), jnp.float32),
    scratch_types=[
        pltpu.VMEM((HV,), jnp.float32),
        pltpu.VMEM((V - HV,), jnp.float32),
        pltpu.VMEM((B,), jnp.float32),
        pltpu.VMEM((ICH,), jnp.int32),
        pltpu.VMEM((ICH,), jnp.int32),
        pltpu.SemaphoreType.DMA,
        pltpu.SemaphoreType.DMA,
        pltpu.SemaphoreType.DMA,
        pltpu.SemaphoreType.DMA,
    ],
    compiler_params=pltpu.CompilerParams(needs_layout_passes=False),
)
def _emb_lookup_sum(tabfd, idxT, outT, r0, r1, acc, ib0, ib1,
                    semr0, semr1, semi0, semi1):
    d = lax.axis_index("s") * NC + lax.axis_index("c")

    def zero(i, _):
        acc[pl.ds(i * L, L)] = jnp.zeros((L,), jnp.float32)
        return 0

    lax.fori_loop(0, B // L, zero, 0, unroll=8)

    rows = (r0, r1)
    semrs = (semr0, semr1)
    ibs = (ib0, ib1)
    semis = (semi0, semi1)

    def issue_row(f, h):
        rowref = tabfd.at[f * D + d]
        if h == 0:
            pltpu.async_copy(rowref.at[pl.ds(0, HV)], r0, semr0)
        else:
            pltpu.async_copy(rowref.at[pl.ds(HV, V - HV)], r1, semr1)

    def drain_row(h):
        if h == 0:
            pltpu.make_async_copy(
                tabfd.at[0].at[pl.ds(0, HV)], r0, semr0).wait()
        else:
            pltpu.make_async_copy(
                tabfd.at[0].at[pl.ds(HV, V - HV)], r1, semr1).wait()

    issue_row(0, 0)
    issue_row(0, 1)

    def step_pair(f, _):
        for h in range(2):
            rbuf = rows[h]
            lbase = h * HV
            mbase = h * HV
            mlen = HV if h == 0 else V - HV

            drain_row(h)

            pltpu.async_copy(idxT.at[f, pl.ds(0, ICH)], ib0, semi0)

            for c in range(NICH):
                p = c % 2
                q = 1 - p
                pltpu.make_async_copy(
                    idxT.at[f, pl.ds(0, ICH)], ibs[p], semis[p]).wait()
                if c + 1 < NICH:
                    pltpu.async_copy(
                        idxT.at[f, pl.ds((c + 1) * ICH, ICH)], ibs[q],
                        semis[q])
                ib = ibs[p]
                cbase = c * ICH

                def body(r, _, ib=ib, cbase=cbase, lbase=lbase,
                         mbase=mbase, mlen=mlen, rbuf=rbuf):
                    iv = ib[pl.ds(r * L, L)]
                    mask = (iv - mbase).astype(jnp.uint32) < jnp.uint32(mlen)
                    loc = iv - lbase
                    g = plsc.load_gather(rbuf, [loc], mask=mask)
                    g = jnp.where(mask, g, jnp.float32(0.0))
                    plsc.addupdate(acc.at[pl.ds(cbase + r * L, L)], g)
                    return 0

                lax.fori_loop(0, ICH // L, body, 0, unroll=8)

            @pl.when(f + 1 < F)
            def _(f=f, h=h):
                issue_row(f + 1, h)

        return 0

    lax.fori_loop(0, F, step_pair, 0)
    pltpu.sync_copy(acc, outT.at[d])


def kernel(x, tables):
    x = x.astype(jnp.int32)
    xT = x.T
    tabfd = tables.transpose(0, 2, 1).reshape(F * D, V)
    outT = _emb_lookup_sum(tabfd, xT)
    return outT.T


# speedup vs baseline: 1.3354x; 1.3345x over previous
import functools

import jax
import jax.numpy as jnp
from jax import lax
from jax.experimental import pallas as pl
from jax.experimental.pallas import tpu as pltpu
from jax.experimental.pallas import tpu_sc as plsc

F = 26
V = 100000
D = 32
B = 16384

NC = 2
NS = 16
NW = NC * NS
L = 16
ICH = 4096
NICH = B // ICH


@functools.partial(
    pl.kernel,
    mesh=plsc.VectorSubcoreMesh(core_axis_name="c", subcore_axis_name="s"),
    out_type=jax.ShapeDtypeStruct((D, B), jnp.float32),
    scratch_types=[
        pltpu.VMEM((V,), jnp.float32),
        pltpu.VMEM((B,), jnp.float32),
        pltpu.VMEM((ICH,), jnp.int32),
        pltpu.VMEM((ICH,), jnp.int32),
        pltpu.SemaphoreType.DMA,
        pltpu.SemaphoreType.DMA,
        pltpu.SemaphoreType.DMA,
    ],
    compiler_params=pltpu.CompilerParams(needs_layout_passes=False),
)
def _emb_lookup_sum(tabfd, idxT, outT, row, acc, ib0, ib1, semr, semi0, semi1):
    d = lax.axis_index("s") * NC + lax.axis_index("c")

    def zero(i, _):
        acc[pl.ds(i * L, L)] = jnp.zeros((L,), jnp.float32)
        return 0

    lax.fori_loop(0, B // L, zero, 0, unroll=8)

    ibs = (ib0, ib1)
    semis = (semi0, semi1)

    pltpu.async_copy(idxT.at[0, pl.ds(0, ICH)], ib0, semi0)

    def field(f, _):
        pltpu.async_copy(tabfd.at[f * D + d], row, semr).wait()

        for c in range(NICH):
            p = c % 2
            q = 1 - p
            pltpu.make_async_copy(
                idxT.at[f, pl.ds(0, ICH)], ibs[p], semis[p]).wait()
            if c + 1 < NICH:
                pltpu.async_copy(
                    idxT.at[f, pl.ds((c + 1) * ICH, ICH)], ibs[q], semis[q])
            else:
                @pl.when(f + 1 < F)
                def _(f=f, q=q):
                    pltpu.async_copy(
                        idxT.at[f + 1, pl.ds(0, ICH)], ibs[q], semis[q])
            ib = ibs[p]
            base = c * ICH

            def body(r, _, ib=ib, base=base):
                iv = ib[pl.ds(r * L, L)]
                g = plsc.load_gather(row, [iv])
                plsc.addupdate(acc.at[pl.ds(base + r * L, L)], g)
                return 0

            lax.fori_loop(0, ICH // L, body, 0, unroll=16)
        return 0

    lax.fori_loop(0, F, field, 0)
    pltpu.sync_copy(acc, outT.at[d])


def kernel(x, tables):
    x = x.astype(jnp.int32)
    xT = x.T
    tabfd = tables.transpose(0, 2, 1).reshape(F * D, V)
    outT = _emb_lookup_sum(tabfd, xT)
    return outT.T


# speedup vs baseline: 1.5180x; 1.1367x over previous
import functools

import jax
import jax.numpy as jnp
from jax import lax
from jax.experimental import pallas as pl
from jax.experimental.pallas import tpu as pltpu
from jax.experimental.pallas import tpu_sc as plsc

F = 26
V = 100000
D = 32
B = 16384

NC = 2
NS = 16
NW = NC * NS
L = 16
ICH = 4096
NICH = B // ICH


@functools.partial(
    pl.kernel,
    mesh=plsc.VectorSubcoreMesh(core_axis_name="c", subcore_axis_name="s"),
    out_type=jax.ShapeDtypeStruct((D, B), jnp.float32),
    scratch_types=[
        pltpu.VMEM((V,), jnp.float32),
        pltpu.VMEM((B,), jnp.float32),
        pltpu.VMEM((ICH,), jnp.int32),
        pltpu.VMEM((ICH,), jnp.int32),
        pltpu.SemaphoreType.DMA,
        pltpu.SemaphoreType.DMA,
        pltpu.SemaphoreType.DMA,
    ],
    compiler_params=pltpu.CompilerParams(needs_layout_passes=False),
)
def _emb_lookup_sum(tabfd, idxT, outT, row, acc, ib0, ib1, semr, semi0, semi1):
    d = lax.axis_index("s") * NC + lax.axis_index("c")

    def zero(i, _):
        acc[pl.ds(i * L, L)] = jnp.zeros((L,), jnp.float32)
        return 0

    lax.fori_loop(0, B // L, zero, 0, unroll=8)

    ibs = (ib0, ib1)
    semis = (semi0, semi1)

    pltpu.async_copy(idxT.at[0, pl.ds(0, ICH)], ib0, semi0)

    def field(f, _):
        pltpu.async_copy(tabfd.at[f * D + d], row, semr).wait()

        for c in range(NICH):
            p = c % 2
            q = 1 - p
            pltpu.make_async_copy(
                idxT.at[f, pl.ds(0, ICH)], ibs[p], semis[p]).wait()
            if c + 1 < NICH:
                pltpu.async_copy(
                    idxT.at[f, pl.ds((c + 1) * ICH, ICH)], ibs[q], semis[q])
            else:
                @pl.when(f + 1 < F)
                def _(f=f, q=q):
                    pltpu.async_copy(
                        idxT.at[f + 1, pl.ds(0, ICH)], ibs[q], semis[q])
            ib = ibs[p]
            base = c * ICH

            @plsc.parallel_loop(0, ICH // L, unroll=16)
            def _(r, ib=ib, base=base):
                iv = ib[pl.ds(r * L, L)]
                g = plsc.load_gather(row, [iv])
                plsc.addupdate(acc.at[pl.ds(base + r * L, L)], g)
        return 0

    lax.fori_loop(0, F, field, 0)
    pltpu.sync_copy(acc, outT.at[d])


def kernel(x, tables):
    x = x.astype(jnp.int32)
    xT = x.T
    tabfd = tables.transpose(0, 2, 1).reshape(F * D, V)
    outT = _emb_lookup_sum(tabfd, xT)
    return outT.T


# speedup vs baseline: 1.5216x; 1.0024x over previous
import functools

import jax
import jax.numpy as jnp
from jax import lax
from jax.experimental import pallas as pl
from jax.experimental.pallas import tpu as pltpu
from jax.experimental.pallas import tpu_sc as plsc

F = 26
V = 100000
D = 32
B = 16384

NC = 2
NS = 16
NW = NC * NS
L = 16
ICH = 4096
NICH = B // ICH


@functools.partial(
    pl.kernel,
    mesh=plsc.VectorSubcoreMesh(core_axis_name="c", subcore_axis_name="s"),
    out_type=jax.ShapeDtypeStruct((D, B), jnp.float32),
    scratch_types=[
        pltpu.VMEM((V,), jnp.float32),
        pltpu.VMEM((B,), jnp.float32),
        pltpu.VMEM((ICH,), jnp.int32),
        pltpu.VMEM((ICH,), jnp.int32),
        pltpu.SemaphoreType.DMA,
        pltpu.SemaphoreType.DMA,
        pltpu.SemaphoreType.DMA,
    ],
    compiler_params=pltpu.CompilerParams(needs_layout_passes=False),
)
def _emb_lookup_sum(tabfd, idxT, outT, row, acc, ib0, ib1, semr, semi0, semi1):
    d = lax.axis_index("s") * NC + lax.axis_index("c")

    @plsc.parallel_loop(0, B // L, unroll=8)
    def _(i):
        acc[pl.ds(i * L, L)] = jnp.zeros((L,), jnp.float32)

    ibs = (ib0, ib1)
    semis = (semi0, semi1)

    pltpu.async_copy(idxT.at[0, pl.ds(0, ICH)], ib0, semi0)

    def field(f, _):
        pltpu.async_copy(tabfd.at[f * D + d], row, semr).wait()

        for c in range(NICH):
            p = c % 2
            q = 1 - p
            pltpu.make_async_copy(
                idxT.at[f, pl.ds(0, ICH)], ibs[p], semis[p]).wait()
            if c + 1 < NICH:
                pltpu.async_copy(
                    idxT.at[f, pl.ds((c + 1) * ICH, ICH)], ibs[q], semis[q])
            else:
                @pl.when(f + 1 < F)
                def _(f=f, q=q):
                    pltpu.async_copy(
                        idxT.at[f + 1, pl.ds(0, ICH)], ibs[q], semis[q])
            ib = ibs[p]
            base = c * ICH

            @plsc.parallel_loop(0, ICH // L, unroll=16)
            def _(r, ib=ib, base=base):
                iv = ib[pl.ds(r * L, L)]
                g = plsc.load_gather(row, [iv])
                plsc.addupdate(acc.at[pl.ds(base + r * L, L)], g)
        return 0

    lax.fori_loop(0, F, field, 0)
    pltpu.sync_copy(acc, outT.at[d])


def kernel(x, tables):
    x = x.astype(jnp.int32)
    xT = x.T
    tabfd = tables.transpose(0, 2, 1).reshape(F * D, V)
    outT = _emb_lookup_sum(tabfd, xT)
    return outT.T
